# Initial kernel scaffold; baseline (speedup 1.0000x reference)
import jax
import jax.numpy as jnp
from jax.experimental import pallas as pl


def kernel(x, Wr, br, W1, b1, W2, b2):
    raise NotImplementedError("write your pallas kernel here")


# speedup vs baseline: 1.0593x; 1.0593x over previous
import functools

import jax
import jax.numpy as jnp
from jax.experimental import pallas as pl
from jax.experimental.pallas import tpu as pltpu

_TOK_TILE = 512


def _moe_dense_body(x_ref, wr_ref, br_ref, w1_ref, b1_ref, w2_ref, b2_ref,
                    out_ref):
    e = pl.program_id(1)
    n_e = pl.num_programs(1)
    x = x_ref[...]
    logits = jnp.dot(x, wr_ref[...], preferred_element_type=jnp.float32)
    logits = logits + br_ref[...]
    iota = jax.lax.broadcasted_iota(jnp.int32, logits.shape, 1)
    v1 = jnp.max(logits, axis=1, keepdims=True)
    i1 = jnp.min(jnp.where(logits == v1, iota, n_e), axis=1, keepdims=True)
    m1 = iota == i1
    masked = jnp.where(m1, -jnp.inf, logits)
    v2 = jnp.max(masked, axis=1, keepdims=True)
    i2 = jnp.min(jnp.where(masked == v2, iota, n_e), axis=1, keepdims=True)
    m2 = iota == i2
    e2 = jnp.exp(v2 - v1)
    denom = 1.0 + e2
    g = jnp.where(m1, 1.0 / denom, 0.0) + jnp.where(m2, e2 / denom, 0.0)
    ge = jnp.sum(jnp.where(iota == e, g, 0.0), axis=1)

    h = jnp.dot(x, w1_ref[0], preferred_element_type=jnp.float32)
    h = jnp.maximum(h + b1_ref[0], 0.0)
    oe = jnp.dot(h, w2_ref[0], preferred_element_type=jnp.float32)
    oe = oe + b2_ref[0]
    contrib = ge[:, None] * oe

    @pl.when(e == 0)
    def _():
        out_ref[...] = contrib

    @pl.when(e != 0)
    def _():
        out_ref[...] += contrib


def _moe_dense(tokens, Wr, br, W1, b1, W2, b2):
    T, H = tokens.shape
    E = Wr.shape[1]
    INNER = W1.shape[2]
    grid = (T // _TOK_TILE, E)
    return pl.pallas_call(
        _moe_dense_body,
        grid=grid,
        in_specs=[
            pl.BlockSpec((_TOK_TILE, H), lambda t, e: (t, 0)),
            pl.BlockSpec((H, E), lambda t, e: (0, 0)),
            pl.BlockSpec((1, E), lambda t, e: (0, 0)),
            pl.BlockSpec((1, H, INNER), lambda t, e: (e, 0, 0)),
            pl.BlockSpec((1, 1, INNER), lambda t, e: (e, 0, 0)),
            pl.BlockSpec((1, INNER, H), lambda t, e: (e, 0, 0)),
            pl.BlockSpec((1, 1, H), lambda t, e: (e, 0, 0)),
        ],
        out_specs=pl.BlockSpec((_TOK_TILE, H), lambda t, e: (t, 0)),
        out_shape=jax.ShapeDtypeStruct((T, H), jnp.float32),
    )(tokens, Wr, br.reshape(1, E), W1, b1.reshape(E, 1, INNER), W2,
      b2.reshape(E, 1, H))


def kernel(x, Wr, br, W1, b1, W2, b2):
    B, S, H = x.shape
    tokens = x.reshape(-1, H)
    out = _moe_dense(tokens, Wr, br, W1, b1, W2, b2)
    return out.reshape(B, S, H)


# speedup vs baseline: 1.0932x; 1.0320x over previous
import functools

import jax
import jax.numpy as jnp
from jax.experimental import pallas as pl
from jax.experimental.pallas import tpu as pltpu

_TILE = 256
_RT_TILE = 1024


def _router_body(x_ref, wr_ref, br_ref, idx_ref, w_ref):
    x = x_ref[...]
    logits = jnp.dot(x, wr_ref[...], preferred_element_type=jnp.float32)
    logits = logits + br_ref[...]
    n_e = logits.shape[1]
    iota = jax.lax.broadcasted_iota(jnp.int32, logits.shape, 1)
    v1 = jnp.max(logits, axis=1, keepdims=True)
    i1 = jnp.min(jnp.where(logits == v1, iota, n_e), axis=1, keepdims=True)
    masked = jnp.where(iota == i1, -jnp.inf, logits)
    v2 = jnp.max(masked, axis=1, keepdims=True)
    i2 = jnp.min(jnp.where(masked == v2, iota, n_e), axis=1, keepdims=True)
    e2 = jnp.exp(v2 - v1)
    denom = 1.0 + e2
    idx_ref[...] = jnp.concatenate([i1, i2], axis=1)
    w_ref[...] = jnp.concatenate([1.0 / denom, e2 / denom], axis=1)


def _router(tokens, Wr, br):
    T, H = tokens.shape
    E = Wr.shape[1]
    grid = (T // _RT_TILE,)
    return pl.pallas_call(
        _router_body,
        grid=grid,
        in_specs=[
            pl.BlockSpec((_RT_TILE, H), lambda t: (t, 0)),
            pl.BlockSpec((H, E), lambda t: (0, 0)),
            pl.BlockSpec((1, E), lambda t: (0, 0)),
        ],
        out_specs=[
            pl.BlockSpec((_RT_TILE, 2), lambda t: (t, 0)),
            pl.BlockSpec((_RT_TILE, 2), lambda t: (t, 0)),
        ],
        out_shape=[
            jax.ShapeDtypeStruct((T, 2), jnp.int32),
            jax.ShapeDtypeStruct((T, 2), jnp.float32),
        ],
    )(tokens, Wr, br.reshape(1, E))


def _ffn_body(meta_ref, xs_ref, w1_ref, b1_ref, w2_ref, b2_ref, ys_ref):
    t = pl.program_id(0)
    ntiles = pl.num_programs(0)
    valid = meta_ref[ntiles + t]

    @pl.when(valid == 1)
    def _():
        x = xs_ref[...]
        h = jnp.dot(x, w1_ref[0], preferred_element_type=jnp.float32)
        h = jnp.maximum(h + b1_ref[0], 0.0)
        o = jnp.dot(h, w2_ref[0], preferred_element_type=jnp.float32)
        ys_ref[...] = o + b2_ref[0]


def _ffn(xs, meta, W1, b1, W2, b2, ntiles):
    NPAD, H = xs.shape
    E, _, INNER = W1.shape
    grid_spec = pltpu.PrefetchScalarGridSpec(
        num_scalar_prefetch=1,
        grid=(ntiles,),
        in_specs=[
            pl.BlockSpec((_TILE, H), lambda t, m: (t, 0)),
            pl.BlockSpec((1, H, INNER), lambda t, m: (m[t], 0, 0)),
            pl.BlockSpec((1, 1, INNER), lambda t, m: (m[t], 0, 0)),
            pl.BlockSpec((1, INNER, H), lambda t, m: (m[t], 0, 0)),
            pl.BlockSpec((1, 1, H), lambda t, m: (m[t], 0, 0)),
        ],
        out_specs=pl.BlockSpec((_TILE, H), lambda t, m: (t, 0)),
    )
    return pl.pallas_call(
        _ffn_body,
        grid_spec=grid_spec,
        out_shape=jax.ShapeDtypeStruct((NPAD, H), jnp.float32),
    )(meta, xs, W1, b1.reshape(E, 1, INNER), W2, b2.reshape(E, 1, H))


def kernel(x, Wr, br, W1, b1, W2, b2):
    B, S, H = x.shape
    E = Wr.shape[1]
    K = 2
    tokens = x.reshape(-1, H)
    T = tokens.shape[0]
    NTILES = (T * K + E * (_TILE - 1) + _TILE - 1) // _TILE
    NPAD = NTILES * _TILE

    top_idx, top_w = _router(tokens, Wr, br)

    ex = top_idx.reshape(-1)
    oh = jax.nn.one_hot(ex, E, dtype=jnp.int32)
    counts = jnp.sum(oh, axis=0)
    ranks_all = jnp.cumsum(oh, axis=0) - oh
    rank = jnp.sum(ranks_all * oh, axis=1)
    sizes = ((counts + _TILE - 1) // _TILE) * _TILE
    P = jnp.concatenate([jnp.zeros((1,), jnp.int32),
                         jnp.cumsum(sizes)]).astype(jnp.int32)
    pos = P[ex] + rank
    tok = jnp.arange(T * K, dtype=jnp.int32) // K
    xs = jnp.zeros((NPAD, H), jnp.float32).at[pos].set(tokens[tok])
    tile_expert = jnp.clip(
        jnp.searchsorted(P, jnp.arange(NTILES, dtype=jnp.int32) * _TILE,
                         side="right") - 1, 0, E - 1).astype(jnp.int32)
    tile_valid = (jnp.arange(NTILES, dtype=jnp.int32) * _TILE
                  < P[E]).astype(jnp.int32)
    meta = jnp.concatenate([tile_expert, tile_valid])

    ys = _ffn(xs, meta, W1, b1, W2, b2, NTILES)

    posk = pos.reshape(T, K)
    out = (top_w[:, 0:1] * ys[posk[:, 0]] + top_w[:, 1:2] * ys[posk[:, 1]])
    return out.reshape(B, S, H)
